# baseline (speedup 1.0000x reference)
import functools

import jax
import jax.numpy as jnp
from jax import lax
from jax.experimental import pallas as pl
from jax.experimental.pallas import tpu as pltpu
from jax.experimental.pallas import tpu_sc as plsc

_LANES = 128
_NUM_CLUSTERS = 100
_TC_BLK = 2048
_NW = 32
_CH = 256


def _sc_gather(user_indices, item_indices, emb_user_w, emb_item_w,
               user_reprs_w):
    B = user_indices.shape[0]
    b_per_w = B // _NW
    n_ch = b_per_w // _CH
    mesh = plsc.VectorSubcoreMesh(core_axis_name="c", subcore_axis_name="s")

    @functools.partial(
        pl.kernel,
        mesh=mesh,
        out_type=[
            jax.ShapeDtypeStruct((B, 128), jnp.float32),
            jax.ShapeDtypeStruct((B, 128), jnp.float32),
            jax.ShapeDtypeStruct((B, 128), jnp.float32),
        ],
        scratch_types=[
            pltpu.VMEM((b_per_w,), jnp.int32),
            pltpu.VMEM((b_per_w,), jnp.int32),
            pltpu.VMEM((b_per_w,), jnp.int32),
            pltpu.VMEM((b_per_w,), jnp.int32),
            pltpu.VMEM((_CH, 128), jnp.float32),
            pltpu.VMEM((_CH, 128), jnp.float32),
            pltpu.VMEM((_CH, 128), jnp.float32),
            pltpu.SemaphoreType.DMA,
            pltpu.SemaphoreType.DMA,
            pltpu.SemaphoreType.DMA,
        ],
    )
    def k(uidx_hbm, iidx_hbm, emb_u_hbm, emb_i_hbm, reprs_hbm,
          u_out, r_out, it_out,
          uidx_v, iidx_v, uq_v, iq_v, u_v, r_v, it_v, s1, s2, s3):
        wid = lax.axis_index("s") * 2 + lax.axis_index("c")
        base = wid * b_per_w
        pltpu.sync_copy(uidx_hbm.at[pl.ds(base, b_per_w)], uidx_v)
        pltpu.sync_copy(iidx_hbm.at[pl.ds(base, b_per_w)], iidx_v)

        def shift_body(i, carry):
            sl = pl.ds(i * 16, 16)
            uq_v[sl] = lax.shift_right_logical(uidx_v[sl], 2)
            iq_v[sl] = lax.shift_right_logical(iidx_v[sl], 2)
            return carry

        lax.fori_loop(0, b_per_w // 16, shift_body, 0)

        for h in range(n_ch):
            sl = pl.ds(h * _CH, _CH)
            c1 = pltpu.async_copy(emb_u_hbm.at[uq_v.at[sl]], u_v, s1)
            c2 = pltpu.async_copy(reprs_hbm.at[uq_v.at[sl]], r_v, s2)
            c3 = pltpu.async_copy(emb_i_hbm.at[iq_v.at[sl]], it_v, s3)
            c1.wait()
            c2.wait()
            c3.wait()
            osl = pl.ds(base + h * _CH, _CH)
            pltpu.sync_copy(u_v, u_out.at[osl])
            pltpu.sync_copy(r_v, r_out.at[osl])
            pltpu.sync_copy(it_v, it_out.at[osl])

    return k(user_indices, item_indices, emb_user_w, emb_item_w,
             user_reprs_w)


def _extract(wide, off, D):
    acc = wide[:, 0:D]
    for o in range(1, 4):
        acc = jnp.where(off == o, wide[:, o * D:(o + 1) * D], acc)
    return acc


def _tc_body(uw_ref, rw_ref, itw_ref, uidx_ref, iidx_ref, cent_t_ref,
             cent_ref, w_ref, b_ref, out_ref):
    D = cent_t_ref.shape[0]
    uoff = lax.bitwise_and(uidx_ref[...], 3)
    ioff = lax.bitwise_and(iidx_ref[...], 3)
    u = _extract(uw_ref[...], uoff, D)
    r = _extract(rw_ref[...], uoff, D)
    it = _extract(itw_ref[...], ioff, D)
    cent_t = cent_t_ref[...]
    scores = jnp.dot(r, cent_t, preferred_element_type=jnp.float32)
    cnorm = jnp.sum(cent_t * cent_t, axis=0, keepdims=True)
    scores = scores - 0.5 * cnorm
    cid = lax.broadcasted_iota(jnp.int32, scores.shape, 1)
    scores = jnp.where(cid < _NUM_CLUSTERS, scores, -jnp.inf)
    m = jnp.max(scores, axis=1, keepdims=True)
    nearest = jnp.min(jnp.where(scores == m, cid, _LANES), axis=1,
                      keepdims=True)
    onehot = (cid == nearest).astype(jnp.float32)
    proto = jnp.dot(onehot, cent_ref[...],
                    preferred_element_type=jnp.float32)
    prod = u * proto * it
    logits = jnp.sum(prod * w_ref[...], axis=1, keepdims=True) + b_ref[...]
    out_ref[...] = jax.nn.sigmoid(logits)


def _tc_forward(u_wide, r_wide, it_wide, uidx, iidx, cluster_centers, W, b):
    B = u_wide.shape[0]
    C, D = cluster_centers.shape
    cent = jnp.zeros((_LANES, D), jnp.float32).at[:C].set(cluster_centers)
    cent_t = cent.T
    w_row = W.reshape(1, D)
    b_11 = b.reshape(1, 1)
    uidx2 = uidx.reshape(B, 1)
    iidx2 = iidx.reshape(B, 1)
    blk = min(_TC_BLK, B)
    grid = (B // blk,)
    return pl.pallas_call(
        _tc_body,
        grid=grid,
        in_specs=[
            pl.BlockSpec((blk, 128), lambda g: (g, 0)),
            pl.BlockSpec((blk, 128), lambda g: (g, 0)),
            pl.BlockSpec((blk, 128), lambda g: (g, 0)),
            pl.BlockSpec((blk, 1), lambda g: (g, 0)),
            pl.BlockSpec((blk, 1), lambda g: (g, 0)),
            pl.BlockSpec((D, _LANES), lambda g: (0, 0)),
            pl.BlockSpec((_LANES, D), lambda g: (0, 0)),
            pl.BlockSpec((1, D), lambda g: (0, 0)),
            pl.BlockSpec((1, 1), lambda g: (0, 0)),
        ],
        out_specs=pl.BlockSpec((blk, 1), lambda g: (g, 0)),
        out_shape=jax.ShapeDtypeStruct((B, 1), jnp.float32),
    )(u_wide, r_wide, it_wide, uidx2, iidx2, cent_t, cent, w_row, b_11)


def kernel(user_indices, item_indices, emb_user, emb_item, user_reprs,
           cluster_centers, W, b):
    V, D = emb_user.shape
    emb_user_w = emb_user.reshape(V * D // 128, 128)
    emb_item_w = emb_item.reshape(V * D // 128, 128)
    user_reprs_w = user_reprs.reshape(V * D // 128, 128)
    u_wide, r_wide, it_wide = _sc_gather(user_indices, item_indices,
                                         emb_user_w, emb_item_w,
                                         user_reprs_w)
    return _tc_forward(u_wide, r_wide, it_wide, user_indices, item_indices,
                       cluster_centers, W, b)


# speedup vs baseline: 1.6604x; 1.6604x over previous
import functools

import jax
import jax.numpy as jnp
from jax import lax
from jax.experimental import pallas as pl
from jax.experimental.pallas import tpu as pltpu
from jax.experimental.pallas import tpu_sc as plsc

_LANES = 128
_NUM_CLUSTERS = 100
_TC_BLK = 2048
_NW = 32
_CH = 256
_TL = 16384


def _transpose_body(xu_ref, xi_ref, xr_ref, ou_ref, oi_ref, or_ref):
    Q = _TL // 4
    for x_ref, o_ref in ((xu_ref, ou_ref), (xi_ref, oi_ref), (xr_ref, or_ref)):
        y = x_ref[...].T
        for k in range(4):
            o_ref[:, k * 32:(k + 1) * 32] = y[k * Q:(k + 1) * Q, :]


def _tc_transpose3(ut, it, rt):
    D, V = ut.shape
    n_g = pl.cdiv(V, _TL)
    wide = jax.ShapeDtypeStruct((n_g * _TL // 4, 128), jnp.float32)
    return pl.pallas_call(
        _transpose_body,
        grid=(n_g,),
        in_specs=[pl.BlockSpec((D, _TL), lambda g: (0, g))] * 3,
        out_specs=[pl.BlockSpec((_TL // 4, 128), lambda g: (g, 0))] * 3,
        out_shape=[wide, wide, wide],
    )(ut, it, rt)


def _sc_gather(user_indices, item_indices, emb_user_w, emb_item_w,
               user_reprs_w):
    B = user_indices.shape[0]
    b_per_w = B // _NW
    n_ch = b_per_w // _CH
    mesh = plsc.VectorSubcoreMesh(core_axis_name="c", subcore_axis_name="s")

    @functools.partial(
        pl.kernel,
        mesh=mesh,
        out_type=[
            jax.ShapeDtypeStruct((B, 128), jnp.float32),
            jax.ShapeDtypeStruct((B, 128), jnp.float32),
            jax.ShapeDtypeStruct((B, 128), jnp.float32),
        ],
        scratch_types=[
            pltpu.VMEM((b_per_w,), jnp.int32),
            pltpu.VMEM((b_per_w,), jnp.int32),
            pltpu.VMEM((b_per_w,), jnp.int32),
            pltpu.VMEM((b_per_w,), jnp.int32),
            pltpu.VMEM((_CH, 128), jnp.float32),
            pltpu.VMEM((_CH, 128), jnp.float32),
            pltpu.VMEM((_CH, 128), jnp.float32),
            pltpu.SemaphoreType.DMA,
            pltpu.SemaphoreType.DMA,
            pltpu.SemaphoreType.DMA,
        ],
    )
    def k(uidx_hbm, iidx_hbm, emb_u_hbm, emb_i_hbm, reprs_hbm,
          u_out, r_out, it_out,
          uidx_v, iidx_v, uq_v, iq_v, u_v, r_v, it_v, s1, s2, s3):
        wid = lax.axis_index("s") * 2 + lax.axis_index("c")
        base = wid * b_per_w
        pltpu.sync_copy(uidx_hbm.at[pl.ds(base, b_per_w)], uidx_v)
        pltpu.sync_copy(iidx_hbm.at[pl.ds(base, b_per_w)], iidx_v)

        def wrow(v):
            hi = lax.shift_left(lax.shift_right_logical(v, 14), 12)
            return hi + lax.bitwise_and(v, 4095)

        def shift_body(i, carry):
            sl = pl.ds(i * 16, 16)
            uq_v[sl] = wrow(uidx_v[sl])
            iq_v[sl] = wrow(iidx_v[sl])
            return carry

        lax.fori_loop(0, b_per_w // 16, shift_body, 0)

        for h in range(n_ch):
            sl = pl.ds(h * _CH, _CH)
            c1 = pltpu.async_copy(emb_u_hbm.at[uq_v.at[sl]], u_v, s1)
            c2 = pltpu.async_copy(reprs_hbm.at[uq_v.at[sl]], r_v, s2)
            c3 = pltpu.async_copy(emb_i_hbm.at[iq_v.at[sl]], it_v, s3)
            c1.wait()
            c2.wait()
            c3.wait()
            osl = pl.ds(base + h * _CH, _CH)
            pltpu.sync_copy(u_v, u_out.at[osl])
            pltpu.sync_copy(r_v, r_out.at[osl])
            pltpu.sync_copy(it_v, it_out.at[osl])

    return k(user_indices, item_indices, emb_user_w, emb_item_w,
             user_reprs_w)


def _extract(wide, off, D):
    acc = wide[:, 0:D]
    for o in range(1, 4):
        acc = jnp.where(off == o, wide[:, o * D:(o + 1) * D], acc)
    return acc


def _tc_body(uw_ref, rw_ref, itw_ref, uidx_ref, iidx_ref, cent_t_ref,
             cent_ref, w_ref, b_ref, out_ref):
    D = cent_t_ref.shape[0]
    uoff = lax.shift_right_logical(
        lax.bitwise_and(uidx_ref[...], _TL - 1), 12)
    ioff = lax.shift_right_logical(
        lax.bitwise_and(iidx_ref[...], _TL - 1), 12)
    u = _extract(uw_ref[...], uoff, D)
    r = _extract(rw_ref[...], uoff, D)
    it = _extract(itw_ref[...], ioff, D)
    cent_t = cent_t_ref[...]
    scores = jnp.dot(r, cent_t, preferred_element_type=jnp.float32)
    cnorm = jnp.sum(cent_t * cent_t, axis=0, keepdims=True)
    scores = scores - 0.5 * cnorm
    cid = lax.broadcasted_iota(jnp.int32, scores.shape, 1)
    scores = jnp.where(cid < _NUM_CLUSTERS, scores, -jnp.inf)
    m = jnp.max(scores, axis=1, keepdims=True)
    nearest = jnp.min(jnp.where(scores == m, cid, _LANES), axis=1,
                      keepdims=True)
    onehot = (cid == nearest).astype(jnp.float32)
    proto = jnp.dot(onehot, cent_ref[...],
                    preferred_element_type=jnp.float32)
    prod = u * proto * it
    logits = jnp.sum(prod * w_ref[...], axis=1, keepdims=True) + b_ref[...]
    out_ref[...] = jax.nn.sigmoid(logits)


def _tc_forward(u_wide, r_wide, it_wide, uidx, iidx, cluster_centers, W, b):
    B = u_wide.shape[0]
    C, D = cluster_centers.shape
    cent = jnp.zeros((_LANES, D), jnp.float32).at[:C].set(cluster_centers)
    cent_t = cent.T
    w_row = W.reshape(1, D)
    b_11 = b.reshape(1, 1)
    uidx2 = uidx.reshape(B, 1)
    iidx2 = iidx.reshape(B, 1)
    blk = min(_TC_BLK, B)
    grid = (B // blk,)
    return pl.pallas_call(
        _tc_body,
        grid=grid,
        in_specs=[
            pl.BlockSpec((blk, 128), lambda g: (g, 0)),
            pl.BlockSpec((blk, 128), lambda g: (g, 0)),
            pl.BlockSpec((blk, 128), lambda g: (g, 0)),
            pl.BlockSpec((blk, 1), lambda g: (g, 0)),
            pl.BlockSpec((blk, 1), lambda g: (g, 0)),
            pl.BlockSpec((D, _LANES), lambda g: (0, 0)),
            pl.BlockSpec((_LANES, D), lambda g: (0, 0)),
            pl.BlockSpec((1, D), lambda g: (0, 0)),
            pl.BlockSpec((1, 1), lambda g: (0, 0)),
        ],
        out_specs=pl.BlockSpec((blk, 1), lambda g: (g, 0)),
        out_shape=jax.ShapeDtypeStruct((B, 1), jnp.float32),
    )(u_wide, r_wide, it_wide, uidx2, iidx2, cent_t, cent, w_row, b_11)


def kernel(user_indices, item_indices, emb_user, emb_item, user_reprs,
           cluster_centers, W, b):
    emb_user_w, emb_item_w, user_reprs_w = _tc_transpose3(
        emb_user.T, emb_item.T, user_reprs.T)
    u_wide, r_wide, it_wide = _sc_gather(user_indices, item_indices,
                                         emb_user_w, emb_item_w,
                                         user_reprs_w)
    return _tc_forward(u_wide, r_wide, it_wide, user_indices, item_indices,
                       cluster_centers, W, b)
